# baseline (speedup 1.0000x reference)
import jax
import jax.numpy as jnp
from jax.experimental import pallas as pl
from jax.experimental.pallas import tpu as pltpu

_PREC = jax.lax.Precision.HIGHEST
_DN = (((1,), (0,)), ((), ()))


def _proj_kernel(x_ref, w1_ref, w4_ref, o1_ref, o4_ref):
    x = x_ref[...]
    o1_ref[...] = jax.lax.dot_general(
        x, w1_ref[...], _DN, precision=_PREC, preferred_element_type=jnp.float32)
    o4_ref[...] = jax.lax.dot_general(
        x, w4_ref[...], _DN, precision=_PREC, preferred_element_type=jnp.float32)


def _initial_supports(x, w1, w4):
    n, _ = x.shape
    f1, f4 = w1.shape[1], w4.shape[1]
    return pl.pallas_call(
        _proj_kernel,
        out_shape=(jax.ShapeDtypeStruct((n, f1), jnp.float32),
                   jax.ShapeDtypeStruct((n, f4), jnp.float32)),
    )(x, w1, w4)


def _agg_kernel(a_ref, s_ref, b_ref, w_ref, c_ref, o_ref):
    h = jax.lax.dot_general(
        a_ref[...], s_ref[...], _DN, precision=_PREC,
        preferred_element_type=jnp.float32)
    h = jnp.maximum(h + b_ref[...], 0.0)
    o_ref[...] = jax.lax.dot_general(
        h, w_ref[...], _DN, precision=_PREC,
        preferred_element_type=jnp.float32) + c_ref[...]


def _agg(adj, s, b, w, c, bm):
    n = adj.shape[0]
    f = s.shape[1]
    g = w.shape[1]
    return pl.pallas_call(
        _agg_kernel,
        grid=(n // bm,),
        in_specs=[
            pl.BlockSpec((bm, n), lambda i: (i, 0)),
            pl.BlockSpec((n, f), lambda i: (0, 0)),
            pl.BlockSpec((1, f), lambda i: (0, 0)),
            pl.BlockSpec((f, g), lambda i: (0, 0)),
            pl.BlockSpec((1, g), lambda i: (0, 0)),
        ],
        out_specs=pl.BlockSpec((bm, g), lambda i: (i, 0)),
        out_shape=jax.ShapeDtypeStruct((n, g), jnp.float32),
        compiler_params=pltpu.CompilerParams(
            dimension_semantics=("parallel",)),
    )(adj, s, b, w, c)


def kernel(x, adj1, adj2, adj3, adj4, adj5, y, index,
           W1, b1, W2, b2, W3, b3, W4, b4, W5, b5, W6, b6,
           Wd1, bd1, Wd2, bd2, Wd3, bd3):
    n = x.shape[0]
    bm = 400 if n % 400 == 0 else n

    s1, s4 = _initial_supports(x, W1, W4)

    b1r, b2r, b3r = b1[None, :], b2[None, :], b3[None, :]
    b4r, b5r, b6r = b4[None, :], b5[None, :], b6[None, :]
    bd1r, bd2r, bd3r = bd1[None, :], bd2[None, :], bd3[None, :]
    z3 = jnp.zeros((1, W2.shape[1]), jnp.float32)
    z4 = jnp.zeros((1, W3.shape[1]), jnp.float32)

    t = _agg(adj5, s1, b1r, W2, z3, bm)
    t = _agg(adj5, t, b2r, W3, z4, bm)
    o1 = _agg(adj5, t, b3r, Wd1, bd1r, bm)

    t = _agg(adj4, s4, b4r, W5, z3, bm)
    t = _agg(adj4, t, b5r, W6, z4, bm)
    o2 = _agg(adj4, t, b6r, Wd2, bd2r, bm)

    t = _agg(adj3, s4, b4r, W5, z3, bm)
    t = _agg(adj3, t, b5r, W6, z4, bm)
    o3 = _agg(adj3, t, b6r, Wd3, bd3r, bm)

    return o1 + o2 + o3


# speedup vs baseline: 2.8519x; 2.8519x over previous
import jax
import jax.numpy as jnp
from jax.experimental import pallas as pl
from jax.experimental.pallas import tpu as pltpu

_PREC = jax.lax.Precision.HIGHEST
_DN = (((1,), (0,)), ((), ()))


def _proj_kernel(x_ref, w1_ref, w4_ref, o1_ref, o4_ref):
    x = x_ref[...]
    o1_ref[...] = jax.lax.dot_general(
        x, w1_ref[...], _DN, precision=_PREC, preferred_element_type=jnp.float32)
    o4_ref[...] = jax.lax.dot_general(
        x, w4_ref[...], _DN, precision=_PREC, preferred_element_type=jnp.float32)


def _initial_supports(x, w1, w4):
    n, _ = x.shape
    f1, f4 = w1.shape[1], w4.shape[1]
    return pl.pallas_call(
        _proj_kernel,
        out_shape=(jax.ShapeDtypeStruct((n, f1), jnp.float32),
                   jax.ShapeDtypeStruct((n, f4), jnp.float32)),
    )(x, w1, w4)


def _epilogue(h, b_ref, w_ref, c_ref):
    h = jnp.maximum(h + b_ref[...], 0.0)
    return jax.lax.dot_general(
        h, w_ref[...], _DN, precision=_PREC,
        preferred_element_type=jnp.float32) + c_ref[...]


def _agg_first_kernel(a_ref, s_ref, b_ref, w_ref, c_ref, o_ref, abf_ref):
    a = a_ref[...].astype(jnp.bfloat16)
    abf_ref[...] = a
    h = jax.lax.dot_general(
        a, s_ref[...].astype(jnp.bfloat16), _DN,
        preferred_element_type=jnp.float32)
    o_ref[...] = _epilogue(h, b_ref, w_ref, c_ref)


def _agg_rest_kernel(a_ref, s_ref, b_ref, w_ref, c_ref, o_ref):
    h = jax.lax.dot_general(
        a_ref[...], s_ref[...].astype(jnp.bfloat16), _DN,
        preferred_element_type=jnp.float32)
    o_ref[...] = _epilogue(h, b_ref, w_ref, c_ref)


def _agg_first(adj, s, b, w, c, bm):
    n = adj.shape[0]
    f = s.shape[1]
    g = w.shape[1]
    return pl.pallas_call(
        _agg_first_kernel,
        grid=(n // bm,),
        in_specs=[
            pl.BlockSpec((bm, n), lambda i: (i, 0)),
            pl.BlockSpec((n, f), lambda i: (0, 0)),
            pl.BlockSpec((1, f), lambda i: (0, 0)),
            pl.BlockSpec((f, g), lambda i: (0, 0)),
            pl.BlockSpec((1, g), lambda i: (0, 0)),
        ],
        out_specs=(pl.BlockSpec((bm, g), lambda i: (i, 0)),
                   pl.BlockSpec((bm, n), lambda i: (i, 0))),
        out_shape=(jax.ShapeDtypeStruct((n, g), jnp.float32),
                   jax.ShapeDtypeStruct((n, n), jnp.bfloat16)),
        compiler_params=pltpu.CompilerParams(
            dimension_semantics=("parallel",)),
    )(adj, s, b, w, c)


def _agg_rest(adj_bf, s, b, w, c, bm):
    n = adj_bf.shape[0]
    f = s.shape[1]
    g = w.shape[1]
    return pl.pallas_call(
        _agg_rest_kernel,
        grid=(n // bm,),
        in_specs=[
            pl.BlockSpec((bm, n), lambda i: (i, 0)),
            pl.BlockSpec((n, f), lambda i: (0, 0)),
            pl.BlockSpec((1, f), lambda i: (0, 0)),
            pl.BlockSpec((f, g), lambda i: (0, 0)),
            pl.BlockSpec((1, g), lambda i: (0, 0)),
        ],
        out_specs=pl.BlockSpec((bm, g), lambda i: (i, 0)),
        out_shape=jax.ShapeDtypeStruct((n, g), jnp.float32),
        compiler_params=pltpu.CompilerParams(
            dimension_semantics=("parallel",)),
    )(adj_bf, s, b, w, c)


def _branch(adj, s0, bb1, wn1, z1, bb2, wn2, z2, bb3, wh, bh, bm1, bm2):
    t, adj_bf = _agg_first(adj, s0, bb1, wn1, z1, bm1)
    t = _agg_rest(adj_bf, t, bb2, wn2, z2, bm2)
    return _agg_rest(adj_bf, t, bb3, wh, bh, bm2)


def kernel(x, adj1, adj2, adj3, adj4, adj5, y, index,
           W1, b1, W2, b2, W3, b3, W4, b4, W5, b5, W6, b6,
           Wd1, bd1, Wd2, bd2, Wd3, bd3):
    n = x.shape[0]
    bm1 = 200 if n % 200 == 0 else n
    bm2 = 400 if n % 400 == 0 else n

    s1, s4 = _initial_supports(x, W1, W4)

    b1r, b2r, b3r = b1[None, :], b2[None, :], b3[None, :]
    b4r, b5r, b6r = b4[None, :], b5[None, :], b6[None, :]
    z3 = jnp.zeros((1, W2.shape[1]), jnp.float32)
    z4 = jnp.zeros((1, W3.shape[1]), jnp.float32)

    o1 = _branch(adj5, s1, b1r, W2, z3, b2r, W3, z4, b3r,
                 Wd1, bd1[None, :], bm1, bm2)
    o2 = _branch(adj4, s4, b4r, W5, z3, b5r, W6, z4, b6r,
                 Wd2, bd2[None, :], bm1, bm2)
    o3 = _branch(adj3, s4, b4r, W5, z3, b5r, W6, z4, b6r,
                 Wd3, bd3[None, :], bm1, bm2)

    return o1 + o2 + o3
